# baseline (speedup 1.0000x reference)
import functools

import numpy as np

import jax
import jax.numpy as jnp
from jax.experimental import pallas as pl
from jax.experimental.pallas import tpu as pltpu
from jax.sharding import Mesh, NamedSharding, PartitionSpec as P


def _pool_kernel(x_ref, bias_ref, o_ref, *, inv_hw):
    b = bias_ref[...]
    n_slices = x_ref.shape[1] // 128

    acc0 = jnp.where(x_ref[:, 0:128] > b, 1.0, 0.0)
    acc1 = jnp.where(x_ref[:, 128:256] > b, 1.0, 0.0)
    for j in range(2, n_slices):
        g = jnp.where(x_ref[:, j * 128:(j + 1) * 128] > b, 1.0, 0.0)
        if j % 2 == 0:
            acc0 = acc0 + g
        else:
            acc1 = acc1 + g

    o_ref[...] = jnp.sum(acc0 + acc1, axis=-1, keepdims=True) * inv_hw


def _pool_call(x2, bias2):
    rows, hw = x2.shape
    TM = 128
    return pl.pallas_call(
        functools.partial(_pool_kernel, inv_hw=1.0 / hw),
        out_shape=jax.ShapeDtypeStruct((rows, 1), jnp.float32),
        grid=(pl.cdiv(rows, TM),),
        in_specs=[
            pl.BlockSpec((TM, hw), lambda i: (i, 0)),
            pl.BlockSpec((TM, 1), lambda i: (i, 0)),
        ],
        out_specs=pl.BlockSpec((TM, 1), lambda i: (i, 0)),
        compiler_params=pltpu.CompilerParams(
            dimension_semantics=("parallel",),
        ),
    )(x2, bias2)


def kernel(x, bias):
    B, C, H, W = x.shape
    BC, HW = B * C, H * W
    assert HW % 128 == 0

    x2 = x.reshape(BC, HW)
    bias2 = jnp.tile(bias.astype(x.dtype), B).reshape(BC, 1)

    devs = jax.devices()
    if len(devs) >= 2 and BC % 2 == 0:
        mesh = Mesh(np.asarray(devs[:2]), ("d",))
        sh = NamedSharding(mesh, P("d", None))
        x2 = jax.device_put(x2, sh)
        bias2 = jax.device_put(bias2, sh)
        out2 = jax.shard_map(
            _pool_call, mesh=mesh,
            in_specs=(P("d", None), P("d", None)),
            out_specs=P("d", None),
            check_vma=False,
        )(x2, bias2)
    else:
        out2 = _pool_call(x2, bias2)

    return out2.reshape(B, C, 1, 1)


# speedup vs baseline: 3.8349x; 3.8349x over previous
import functools

import jax
import jax.numpy as jnp
from jax.experimental import pallas as pl
from jax.experimental.pallas import tpu as pltpu


def _pool_kernel(x_ref, bias_ref, o_ref, *, tm, inv_hw):
    i = pl.program_id(0)
    b = bias_ref[pl.ds(i * tm, tm), :]
    n_slices = x_ref.shape[1] // 128

    acc0 = jnp.where(x_ref[:, 0:128] > b, 1.0, 0.0)
    acc1 = jnp.where(x_ref[:, 128:256] > b, 1.0, 0.0)
    for j in range(2, n_slices):
        g = jnp.where(x_ref[:, j * 128:(j + 1) * 128] > b, 1.0, 0.0)
        if j % 2 == 0:
            acc0 = acc0 + g
        else:
            acc1 = acc1 + g

    o_ref[...] = jnp.sum(acc0 + acc1, axis=-1, keepdims=True) * inv_hw


def kernel(x, bias):
    B, C, H, W = x.shape
    BC, HW = B * C, H * W
    assert HW % 128 == 0

    x2 = x.reshape(BC, HW)
    bias2 = jnp.tile(bias.astype(x.dtype), B).reshape(BC, 1)

    TM = 64
    grid = pl.cdiv(BC, TM)

    out2 = pl.pallas_call(
        functools.partial(_pool_kernel, tm=TM, inv_hw=1.0 / HW),
        out_shape=jax.ShapeDtypeStruct((BC, 1), jnp.float32),
        grid=(grid,),
        in_specs=[
            pl.BlockSpec((TM, HW), lambda i: (i, 0)),
            pl.BlockSpec((BC, 1), lambda i: (0, 0)),
        ],
        out_specs=pl.BlockSpec((TM, 1), lambda i: (i, 0)),
        compiler_params=pltpu.CompilerParams(
            dimension_semantics=("parallel",),
        ),
    )(x2, bias2)

    return out2.reshape(B, C, 1, 1)


# speedup vs baseline: 4.0478x; 1.0555x over previous
import functools

import jax
import jax.numpy as jnp
from jax.experimental import pallas as pl
from jax.experimental.pallas import tpu as pltpu


def _pool_kernel(x_ref, bias_ref, o_ref, *, tm, inv_hw):
    i = pl.program_id(0)
    b = bias_ref[pl.ds(i * tm, tm), :]
    n_slices = x_ref.shape[1] // 128

    acc0 = jnp.where(x_ref[:, 0:128] > b, 1.0, 0.0)
    acc1 = jnp.where(x_ref[:, 128:256] > b, 1.0, 0.0)
    for j in range(2, n_slices):
        g = jnp.where(x_ref[:, j * 128:(j + 1) * 128] > b, 1.0, 0.0)
        if j % 2 == 0:
            acc0 = acc0 + g
        else:
            acc1 = acc1 + g

    o_ref[...] = jnp.sum(acc0 + acc1, axis=-1, keepdims=True) * inv_hw


def kernel(x, bias):
    B, C, H, W = x.shape
    BC, HW = B * C, H * W
    assert HW % 128 == 0

    x2 = x.reshape(BC, HW)
    bias2 = jnp.tile(bias.astype(x.dtype), B).reshape(BC, 1)

    TM = 128
    grid = pl.cdiv(BC, TM)

    out2 = pl.pallas_call(
        functools.partial(_pool_kernel, tm=TM, inv_hw=1.0 / HW),
        out_shape=jax.ShapeDtypeStruct((BC, 1), jnp.float32),
        grid=(grid,),
        in_specs=[
            pl.BlockSpec((TM, HW), lambda i: (i, 0)),
            pl.BlockSpec((BC, 1), lambda i: (0, 0)),
        ],
        out_specs=pl.BlockSpec((TM, 1), lambda i: (i, 0)),
        compiler_params=pltpu.CompilerParams(
            dimension_semantics=("parallel",),
        ),
    )(x2, bias2)

    return out2.reshape(B, C, 1, 1)


# speedup vs baseline: 4.0614x; 1.0034x over previous
import functools

import jax
import jax.numpy as jnp
from jax.experimental import pallas as pl
from jax.experimental.pallas import tpu as pltpu


def _pool_kernel(x_ref, bias_ref, o_ref, *, inv_hw):
    b = bias_ref[...]
    n_slices = x_ref.shape[1] // 128

    acc0 = jnp.where(x_ref[:, 0:128] > b, 1.0, 0.0)
    acc1 = jnp.where(x_ref[:, 128:256] > b, 1.0, 0.0)
    for j in range(2, n_slices):
        g = jnp.where(x_ref[:, j * 128:(j + 1) * 128] > b, 1.0, 0.0)
        if j % 2 == 0:
            acc0 = acc0 + g
        else:
            acc1 = acc1 + g

    o_ref[...] = jnp.sum(acc0 + acc1, axis=-1, keepdims=True) * inv_hw


def kernel(x, bias):
    B, C, H, W = x.shape
    BC, HW = B * C, H * W
    assert HW % 128 == 0

    x2 = x.reshape(BC, HW)
    bias2 = jnp.tile(bias.astype(x.dtype), B).reshape(BC, 1)

    TM = 128
    grid = pl.cdiv(BC, TM)

    out2 = pl.pallas_call(
        functools.partial(_pool_kernel, inv_hw=1.0 / HW),
        out_shape=jax.ShapeDtypeStruct((BC, 1), jnp.float32),
        grid=(grid,),
        in_specs=[
            pl.BlockSpec((TM, HW), lambda i: (i, 0)),
            pl.BlockSpec((TM, 1), lambda i: (i, 0)),
        ],
        out_specs=pl.BlockSpec((TM, 1), lambda i: (i, 0)),
        compiler_params=pltpu.CompilerParams(
            dimension_semantics=("parallel",),
        ),
    )(x2, bias2)

    return out2.reshape(B, C, 1, 1)
